# Initial kernel scaffold; baseline (speedup 1.0000x reference)
import jax
import jax.numpy as jnp
from jax.experimental import pallas as pl


def kernel(inputs, W):
    raise NotImplementedError("write your pallas kernel here")


# speedup vs baseline: 1.7250x; 1.7250x over previous
import jax
import jax.numpy as jnp
from jax.experimental import pallas as pl


SEQ_BLK = 512


def _add_kernel(x_ref, w_ref, o_ref):
    o_ref[...] = x_ref[...] + w_ref[...][None, :, :]


def kernel(inputs, W):
    batch, seq_len, dim = inputs.shape
    grid = (seq_len // SEQ_BLK,)
    return pl.pallas_call(
        _add_kernel,
        grid=grid,
        in_specs=[
            pl.BlockSpec((batch, SEQ_BLK, dim), lambda i: (0, i, 0)),
            pl.BlockSpec((SEQ_BLK, dim), lambda i: (i, 0)),
        ],
        out_specs=pl.BlockSpec((batch, SEQ_BLK, dim), lambda i: (0, i, 0)),
        out_shape=jax.ShapeDtypeStruct((batch, seq_len, dim), inputs.dtype),
    )(inputs, W)
